# baseline (speedup 1.0000x reference)
import numpy as np

import jax
import jax.numpy as jnp
from jax import lax
from jax.experimental import pallas as pl
from jax.experimental.pallas import tpu as pltpu
from jax.experimental.pallas import tpu_sc as plsc

N_NODES = 10000
D = 128
N_EDGES = 320000

N_TILES = 32
CHUNK = 80
BLK = 21
SC0_BLOCKS = 8
SC1_BLOCKS = 4
EDGES_PER_BLOCK = BLK * CHUNK
E_PAD = 16 * (SC0_BLOCKS + SC1_BLOCKS) * EDGES_PER_BLOCK
SC1_EDGE_BASE = 16 * SC0_BLOCKS * EDGES_PER_BLOCK
ROWS_MAIN = 624
ROWS_LAST = 640

_DMAP = np.concatenate([np.arange(0, D, 2), np.arange(1, D, 2)])
_DMAP2 = _DMAP[_DMAP]


def _sc_agg_body(x_hbm, src_hbm, dst_hbm, w_hbm, out_hbm,
                 idx_a, idx_b, dst_a, dst_b, w_a, w_b, dfl_a, dfl_b,
                 g0, g1, f0, f1, accum_sh, gsem0, gsem1, fsem0, fsem1):
    c = lax.axis_index("c")
    s = lax.axis_index("s")
    base_row = s * ROWS_MAIN
    gbufs = (g0, g1)
    fbufs = (f0, f1)
    gsems = (gsem0, gsem1)
    fsems = (fsem0, fsem1)
    idx_st = (idx_a, idx_b)
    dst_st = (dst_a, dst_b)
    w_st = (w_a, w_b)
    dfl_st = (dfl_a, dfl_b)
    nblk_self = jnp.where(c == 0, SC0_BLOCKS, SC1_BLOCKS)
    nchunk_self = nblk_self * BLK
    edge_base = jnp.where(c == 0, s * (SC0_BLOCKS * EDGES_PER_BLOCK),
                          SC1_EDGE_BASE + s * (SC1_BLOCKS * EDGES_PER_BLOCK))

    def zero_row(i, carry):
        for j in range(8):
            f0[i, pl.ds(j * 16, 16)] = jnp.zeros((16,), jnp.float32)
        return carry
    lax.fori_loop(0, CHUNK, zero_row, 0)
    for t in range(ROWS_MAIN // CHUNK):
        pltpu.sync_copy(f0, accum_sh.at[pl.ds(base_row + t * CHUNK, CHUNK)])

    @pl.when(s < 15)
    def _():
        pltpu.sync_copy(f0.at[pl.ds(0, ROWS_MAIN % CHUNK)],
                        accum_sh.at[pl.ds(base_row + 560, ROWS_MAIN % CHUNK)])

    @pl.when(s == 15)
    def _():
        pltpu.sync_copy(f0, accum_sh.at[pl.ds(base_row + 560, CHUNK)])
    plsc.subcore_barrier()

    def refill_dyn(blk_dyn, sb):
        off = edge_base + blk_dyn * EDGES_PER_BLOCK
        pltpu.sync_copy(src_hbm.at[pl.ds(off, EDGES_PER_BLOCK)], idx_st[sb])
        pltpu.sync_copy(dst_hbm.at[pl.ds(off, EDGES_PER_BLOCK)], dfl_st[sb])
        pltpu.sync_copy(w_hbm.at[pl.ds(off, EDGES_PER_BLOCK)], w_st[sb])
        def relayout(i, carry):
            v = dfl_st[sb][pl.ds(i * 16, 16)]
            dst_st[sb][i // (CHUNK // 16),
                       pl.ds((i % (CHUNK // 16)) * 16, 16)] = v
            return carry
        lax.fori_loop(0, EDGES_PER_BLOCK // 16, relayout, 0)

    def scale_convert(pos, par, p):
        gbuf = gbufs[p]
        fbuf = fbufs[p]

        def grp(g, carry):
            base = g * 16
            wa16 = w_a[pl.ds(pos * CHUNK + base, 16)]
            wb16 = w_b[pl.ds(pos * CHUNK + base, 16)]
            w16 = jnp.where(par == 0, wa16, wb16)
            mask_hi = jnp.full((16,), -65536, jnp.int32)
            for r in range(16):
                wv = jnp.full((16,), w16[r], jnp.float32)
                row = base + r
                for jj in range(4):
                    v16 = gbuf[row, pl.ds(jj * 16, 16)]
                    ev = lax.bitcast_convert_type(v16 << 16, jnp.float32)
                    od = lax.bitcast_convert_type(v16 & mask_hi, jnp.float32)
                    fbuf[row, pl.ds(jj * 16, 16)] = ev * wv
                    fbuf[row, pl.ds(64 + jj * 16, 16)] = od * wv
            return carry
        lax.fori_loop(0, CHUNK // 16, grp, 0)

    refill_dyn(0, 0)
    pltpu.async_copy(x_hbm.at[idx_a.at[pl.ds(0, CHUNK)]], g0, gsems[0])

    def round_body(q, carry):
        k0 = 2 * q
        for b in range(2):
            k = k0 + b
            p = b
            blk = k // BLK
            par = lax.rem(blk, 2)
            pos = lax.rem(k, BLK)

            k1 = k + 1
            pos1 = lax.rem(k1, BLK)
            par1 = lax.rem(k1 // BLK, 2)

            @pl.when((k1 < nchunk_self) & (par1 == 0))
            def _():
                pltpu.async_copy(
                    x_hbm.at[idx_a.at[pl.ds(pos1 * CHUNK, CHUNK)]],
                    gbufs[1 - p], gsems[1 - p])

            @pl.when((k1 < nchunk_self) & (par1 == 1))
            def _():
                pltpu.async_copy(
                    x_hbm.at[idx_b.at[pl.ds(pos1 * CHUNK, CHUNK)]],
                    gbufs[1 - p], gsems[1 - p])

            pltpu.make_async_copy(
                x_hbm.at[idx_a.at[pl.ds(0, CHUNK)]], gbufs[p],
                gsems[p]).wait()
            scale_convert(pos, par, p)

            @pl.when(k > 0)
            def _():
                pltpu.make_async_copy(
                    fbufs[1 - p], accum_sh.at[dst_a.at[0]],
                    fsems[1 - p]).wait()

            @pl.when(par == 0)
            def _():
                pltpu.async_copy(fbufs[p], accum_sh.at[dst_a.at[pos]],
                                 fsems[p], add=True)

            @pl.when(par == 1)
            def _():
                pltpu.async_copy(fbufs[p], accum_sh.at[dst_b.at[pos]],
                                 fsems[p], add=True)

            nblk = blk + 1
            do = (pos == 0) & (nblk < nblk_self)
            rp = lax.rem(nblk, 2)

            @pl.when(do & (rp == 0))
            def _():
                refill_dyn(nblk, 0)

            @pl.when(do & (rp == 1))
            def _():
                refill_dyn(nblk, 1)
        return carry
    lax.fori_loop(0, nchunk_self // 2, round_body, 0)

    pltpu.make_async_copy(
        fbufs[1], accum_sh.at[dst_a.at[0]], fsems[1]).wait()
    plsc.subcore_barrier()

    @pl.when(s < 15)
    def _():
        pltpu.sync_copy(accum_sh.at[pl.ds(base_row, ROWS_MAIN)],
                        out_hbm.at[c, pl.ds(base_row, ROWS_MAIN)])

    @pl.when(s == 15)
    def _():
        pltpu.sync_copy(accum_sh.at[pl.ds(base_row, ROWS_LAST)],
                        out_hbm.at[c, pl.ds(base_row, ROWS_LAST)])


_sc_aggregate = pl.kernel(
    _sc_agg_body,
    out_type=jax.ShapeDtypeStruct((2, N_NODES, D), jnp.float32),
    mesh=plsc.VectorSubcoreMesh(core_axis_name="c", subcore_axis_name="s"),
    compiler_params=pltpu.CompilerParams(use_tc_tiling_on_sc=False),
    scratch_types=[
        pltpu.VMEM((EDGES_PER_BLOCK,), jnp.int32),
        pltpu.VMEM((EDGES_PER_BLOCK,), jnp.int32),
        pltpu.VMEM((BLK, CHUNK), jnp.int32),
        pltpu.VMEM((BLK, CHUNK), jnp.int32),
        pltpu.VMEM((EDGES_PER_BLOCK,), jnp.float32),
        pltpu.VMEM((EDGES_PER_BLOCK,), jnp.float32),
        pltpu.VMEM((EDGES_PER_BLOCK,), jnp.int32),
        pltpu.VMEM((EDGES_PER_BLOCK,), jnp.int32),
        pltpu.VMEM((CHUNK, D // 2), jnp.int32),
        pltpu.VMEM((CHUNK, D // 2), jnp.int32),
        pltpu.VMEM((CHUNK, D), jnp.float32),
        pltpu.VMEM((CHUNK, D), jnp.float32),
        pltpu.VMEM_SHARED((N_NODES, D), jnp.float32),
        pltpu.SemaphoreType.DMA,
        pltpu.SemaphoreType.DMA,
        pltpu.SemaphoreType.DMA,
        pltpu.SemaphoreType.DMA,
    ],
)


def _relu_combine_body(p_ref, b1_ref, o_ref):
    o_ref[...] = jnp.maximum(
        p_ref[0] + p_ref[1] + b1_ref[...], 0.0).astype(jnp.bfloat16)


def _matmul_combine_body(q_ref, w2_ref, b2_ref, o_ref):
    agg = q_ref[0] + q_ref[1]
    o_ref[...] = (
        jnp.dot(agg, w2_ref[...], preferred_element_type=jnp.float32)
        + b2_ref[...])


_TC_BLOCK = 1000


def _relu_combine(p, b1p):
    return pl.pallas_call(
        _relu_combine_body,
        grid=(N_NODES // _TC_BLOCK,),
        in_specs=[
            pl.BlockSpec((2, _TC_BLOCK, D), lambda i: (0, i, 0)),
            pl.BlockSpec((1, D), lambda i: (0, 0)),
        ],
        out_specs=pl.BlockSpec((_TC_BLOCK, D), lambda i: (i, 0)),
        out_shape=jax.ShapeDtypeStruct((N_NODES, D), jnp.bfloat16),
    )(p, b1p.reshape(1, D))


def _matmul_combine(q, W2p, b2):
    return pl.pallas_call(
        _matmul_combine_body,
        grid=(N_NODES // _TC_BLOCK,),
        in_specs=[
            pl.BlockSpec((2, _TC_BLOCK, D), lambda i: (0, i, 0)),
            pl.BlockSpec((D, D), lambda i: (0, 0)),
            pl.BlockSpec((1, D), lambda i: (0, 0)),
        ],
        out_specs=pl.BlockSpec((_TC_BLOCK, D), lambda i: (i, 0)),
        out_shape=jax.ShapeDtypeStruct((N_NODES, D), jnp.float32),
    )(q, W2p, b2.reshape(1, D))


def _prep_edges(edge_index, edge_weight):
    pad = E_PAD - N_EDGES
    return (jnp.pad(edge_index[0].astype(jnp.int32), (0, pad)),
            jnp.pad(edge_index[1].astype(jnp.int32), (0, pad)),
            jnp.pad(edge_weight, (0, pad)))


@jax.jit
def kernel(x, edge_index1, edge_weight1, edge_index2, edge_weight2, W2, b1, b2):
    src1, dst1, w1 = _prep_edges(edge_index1, edge_weight1)
    src2, dst2, w2 = _prep_edges(edge_index2, edge_weight2)

    b1p = b1[_DMAP]
    w2p = W2[_DMAP2, :]

    xi = lax.bitcast_convert_type(
        x.astype(jnp.bfloat16).reshape(N_NODES, D // 2, 2), jnp.int32)
    p1 = _sc_aggregate(xi, src1, dst1, w1)
    h = _relu_combine(p1, b1p)
    hi = lax.bitcast_convert_type(h.reshape(N_NODES, D // 2, 2), jnp.int32)
    p2 = _sc_aggregate(hi, src2, dst2, w2)
    return _matmul_combine(p2, w2p, b2)


# speedup vs baseline: 1.7177x; 1.7177x over previous
import jax
import jax.numpy as jnp
from jax import lax
from jax.experimental import pallas as pl
from jax.experimental.pallas import tpu as pltpu
from jax.experimental.pallas import tpu_sc as plsc

N_NODES = 10000
D = 128
N_EDGES = 320000

N_TILES = 32
CHUNK = 80
BLK = 21
SC0_BLOCKS = 8
SC1_BLOCKS = 4
NBLK_MAX = max(SC0_BLOCKS, SC1_BLOCKS)
EDGES_PER_BLOCK = BLK * CHUNK
E_PAD = 16 * (SC0_BLOCKS + SC1_BLOCKS) * EDGES_PER_BLOCK
SC1_EDGE_BASE = 16 * SC0_BLOCKS * EDGES_PER_BLOCK
ROWS_MAIN = 624
ROWS_LAST = 640


def _sc_agg_body(x_hbm, src_hbm, dst_hbm, w_hbm, out_hbm,
                 idx_a, idx_b, dst_a, dst_b, w_a, w_b, dfl_a, dfl_b,
                 r0, r1, r2, accum_sh, sem0, sem1, sem2):
    c = lax.axis_index("c")
    s = lax.axis_index("s")
    wid = c * 16 + s
    base_row = s * ROWS_MAIN
    rows = (r0, r1, r2)
    sems = (sem0, sem1, sem2)
    idx_st = (idx_a, idx_b)
    dst_st = (dst_a, dst_b)
    w_st = (w_a, w_b)
    dfl_st = (dfl_a, dfl_b)
    nblk_self = jnp.where(c == 0, SC0_BLOCKS, SC1_BLOCKS)
    nchunk_self = nblk_self * BLK
    edge_base = jnp.where(c == 0, s * (SC0_BLOCKS * EDGES_PER_BLOCK),
                          SC1_EDGE_BASE + s * (SC1_BLOCKS * EDGES_PER_BLOCK))

    def zero_row(i, carry):
        for j in range(8):
            r0[i, pl.ds(j * 16, 16)] = jnp.zeros((16,), jnp.float32)
        return carry
    lax.fori_loop(0, CHUNK, zero_row, 0)
    for t in range(ROWS_MAIN // CHUNK):
        pltpu.sync_copy(r0, accum_sh.at[pl.ds(base_row + t * CHUNK, CHUNK)])

    @pl.when(s < 15)
    def _():
        pltpu.sync_copy(r0.at[pl.ds(0, ROWS_MAIN % CHUNK)],
                        accum_sh.at[pl.ds(base_row + 560, ROWS_MAIN % CHUNK)])

    @pl.when(s == 15)
    def _():
        pltpu.sync_copy(r0, accum_sh.at[pl.ds(base_row + 560, CHUNK)])
    plsc.subcore_barrier()

    def refill_dyn(blk_dyn, sb):
        off = edge_base + blk_dyn * EDGES_PER_BLOCK
        pltpu.sync_copy(src_hbm.at[pl.ds(off, EDGES_PER_BLOCK)], idx_st[sb])
        pltpu.sync_copy(dst_hbm.at[pl.ds(off, EDGES_PER_BLOCK)], dfl_st[sb])
        pltpu.sync_copy(w_hbm.at[pl.ds(off, EDGES_PER_BLOCK)], w_st[sb])
        def relayout(i, carry):
            v = dfl_st[sb][pl.ds(i * 16, 16)]
            dst_st[sb][i // (CHUNK // 16),
                       pl.ds((i % (CHUNK // 16)) * 16, 16)] = v
            return carry
        lax.fori_loop(0, EDGES_PER_BLOCK // 16, relayout, 0)

    def scale_dyn(pos, par, m):
        buf = rows[m]

        def grp(g, carry):
            base = g * 16
            wa16 = w_a[pl.ds(pos * CHUNK + base, 16)]
            wb16 = w_b[pl.ds(pos * CHUNK + base, 16)]
            w16 = jnp.where(par == 0, wa16, wb16)
            for r in range(16):
                wv = jnp.full((16,), w16[r], jnp.float32)
                for j in range(8):
                    sl = pl.ds(j * 16, 16)
                    buf[base + r, sl] = buf[base + r, sl] * wv
            return carry
        lax.fori_loop(0, CHUNK // 16, grp, 0)

    refill_dyn(0, 0)
    pltpu.async_copy(x_hbm.at[idx_a.at[pl.ds(0, CHUNK)]], rows[0], sems[0])
    pltpu.async_copy(x_hbm.at[idx_a.at[pl.ds(CHUNK, CHUNK)]], rows[1], sems[1])

    def round_body(q, carry):
        k0 = 3 * q
        for b in range(3):
            k = k0 + b
            m = b
            blk = k // BLK
            par = lax.rem(blk, 2)
            pos = lax.rem(k, BLK)

            pltpu.make_async_copy(
                x_hbm.at[idx_a.at[pl.ds(0, CHUNK)]], rows[m], sems[m]).wait()
            scale_dyn(pos, par, m)

            m_prev = (b + 2) % 3

            @pl.when(k > 0)
            def _():
                pltpu.make_async_copy(
                    rows[m_prev], accum_sh.at[dst_a.at[0]],
                    sems[m_prev]).wait()

            @pl.when(par == 0)
            def _():
                pltpu.async_copy(rows[m], accum_sh.at[dst_a.at[pos]],
                                 sems[m], add=True)

            @pl.when(par == 1)
            def _():
                pltpu.async_copy(rows[m], accum_sh.at[dst_b.at[pos]],
                                 sems[m], add=True)

            k2 = k + 2
            pos2 = lax.rem(k2, BLK)
            par2 = lax.rem(k2 // BLK, 2)

            @pl.when((k2 < nchunk_self) & (par2 == 0))
            def _():
                pltpu.async_copy(
                    x_hbm.at[idx_a.at[pl.ds(pos2 * CHUNK, CHUNK)]],
                    rows[m_prev], sems[m_prev])

            @pl.when((k2 < nchunk_self) & (par2 == 1))
            def _():
                pltpu.async_copy(
                    x_hbm.at[idx_b.at[pl.ds(pos2 * CHUNK, CHUNK)]],
                    rows[m_prev], sems[m_prev])

            if b == 0:
                nblk = blk + 1
                do = (pos == 0) & (nblk < nblk_self)
                rp = lax.rem(nblk, 2)

                @pl.when(do & (rp == 0))
                def _():
                    refill_dyn(nblk, 0)

                @pl.when(do & (rp == 1))
                def _():
                    refill_dyn(nblk, 1)
        return carry
    lax.fori_loop(0, nblk_self * (BLK // 3), round_body, 0)

    pltpu.make_async_copy(
        rows[2], accum_sh.at[dst_a.at[0]], sems[2]).wait()
    plsc.subcore_barrier()

    @pl.when(s < 15)
    def _():
        pltpu.sync_copy(accum_sh.at[pl.ds(base_row, ROWS_MAIN)],
                        out_hbm.at[c, pl.ds(base_row, ROWS_MAIN)])

    @pl.when(s == 15)
    def _():
        pltpu.sync_copy(accum_sh.at[pl.ds(base_row, ROWS_LAST)],
                        out_hbm.at[c, pl.ds(base_row, ROWS_LAST)])


_sc_aggregate = pl.kernel(
    _sc_agg_body,
    out_type=jax.ShapeDtypeStruct((2, N_NODES, D), jnp.float32),
    mesh=plsc.VectorSubcoreMesh(core_axis_name="c", subcore_axis_name="s"),
    scratch_types=[
        pltpu.VMEM((EDGES_PER_BLOCK,), jnp.int32),
        pltpu.VMEM((EDGES_PER_BLOCK,), jnp.int32),
        pltpu.VMEM((BLK, CHUNK), jnp.int32),
        pltpu.VMEM((BLK, CHUNK), jnp.int32),
        pltpu.VMEM((EDGES_PER_BLOCK,), jnp.float32),
        pltpu.VMEM((EDGES_PER_BLOCK,), jnp.float32),
        pltpu.VMEM((EDGES_PER_BLOCK,), jnp.int32),
        pltpu.VMEM((EDGES_PER_BLOCK,), jnp.int32),
        pltpu.VMEM((CHUNK, D), jnp.float32),
        pltpu.VMEM((CHUNK, D), jnp.float32),
        pltpu.VMEM((CHUNK, D), jnp.float32),
        pltpu.VMEM_SHARED((N_NODES, D), jnp.float32),
        pltpu.SemaphoreType.DMA,
        pltpu.SemaphoreType.DMA,
        pltpu.SemaphoreType.DMA,
    ],
)


def _relu_combine_body(p_ref, b1_ref, o_ref):
    o_ref[...] = jnp.maximum(p_ref[0] + p_ref[1] + b1_ref[...], 0.0)


def _matmul_combine_body(q_ref, w2_ref, b2_ref, o_ref):
    agg = q_ref[0] + q_ref[1]
    o_ref[...] = (
        jnp.dot(agg, w2_ref[...], preferred_element_type=jnp.float32)
        + b2_ref[...])


_TC_BLOCK = 1000


def _relu_combine(p, b1):
    return pl.pallas_call(
        _relu_combine_body,
        grid=(N_NODES // _TC_BLOCK,),
        in_specs=[
            pl.BlockSpec((2, _TC_BLOCK, D), lambda i: (0, i, 0)),
            pl.BlockSpec((1, D), lambda i: (0, 0)),
        ],
        out_specs=pl.BlockSpec((_TC_BLOCK, D), lambda i: (i, 0)),
        out_shape=jax.ShapeDtypeStruct((N_NODES, D), jnp.float32),
    )(p, b1.reshape(1, D))


def _matmul_combine(q, W2, b2):
    return pl.pallas_call(
        _matmul_combine_body,
        grid=(N_NODES // _TC_BLOCK,),
        in_specs=[
            pl.BlockSpec((2, _TC_BLOCK, D), lambda i: (0, i, 0)),
            pl.BlockSpec((D, D), lambda i: (0, 0)),
            pl.BlockSpec((1, D), lambda i: (0, 0)),
        ],
        out_specs=pl.BlockSpec((_TC_BLOCK, D), lambda i: (i, 0)),
        out_shape=jax.ShapeDtypeStruct((N_NODES, D), jnp.float32),
    )(q, W2, b2.reshape(1, D))


def _prep_edges(edge_index, edge_weight):
    pad = E_PAD - N_EDGES
    return (jnp.pad(edge_index[0].astype(jnp.int32), (0, pad)),
            jnp.pad(edge_index[1].astype(jnp.int32), (0, pad)),
            jnp.pad(edge_weight, (0, pad)))


@jax.jit
def kernel(x, edge_index1, edge_weight1, edge_index2, edge_weight2, W2, b1, b2):
    src1, dst1, w1 = _prep_edges(edge_index1, edge_weight1)
    src2, dst2, w2 = _prep_edges(edge_index2, edge_weight2)

    p1 = _sc_aggregate(x, src1, dst1, w1)
    h = _relu_combine(p1, b1)
    p2 = _sc_aggregate(h, src2, dst2, w2)
    return _matmul_combine(p2, W2, b2)
